# Initial kernel scaffold; baseline (speedup 1.0000x reference)
import jax
import jax.numpy as jnp
from jax.experimental import pallas as pl


def kernel(input, adj_edge_index, adj_edge_weight, W):
    raise NotImplementedError("write your pallas kernel here")


# speedup vs baseline: 6.5898x; 6.5898x over previous
import jax
import jax.numpy as jnp
from jax import lax
from jax.experimental import pallas as pl
from jax.experimental.pallas import tpu as pltpu
from jax.experimental.pallas import tpu_sc as plsc

N = 10000
E = 320000
D = 128
NC = 2
NS = 16
NW = NC * NS
LANES = 16
NPAD = 10240
RPT = NPAD // NS
C = 80
SB = 25
SBN = 5
ZROWS = 64


def _spmm_body(x_hbm, src_hbm, dst_hbm, w_hbm, partial_hbm,
               src_v, dst_v, w_v, rows_v, zbuf_v, acc_sh, sem):
    cid = lax.axis_index("c")
    sid = lax.axis_index("s")
    wid = cid * NS + sid

    zero16 = jnp.zeros((LANES,), jnp.float32)

    def zrow(j, carry):
        for k in range(D // LANES):
            zbuf_v[j, pl.ds(k * LANES, LANES)] = zero16
        return carry

    lax.fori_loop(0, ZROWS, zrow, 0)
    for k in range(RPT // ZROWS):
        r0 = pl.multiple_of(sid * RPT + k * ZROWS, 8)
        pltpu.sync_copy(zbuf_v, acc_sh.at[pl.ds(r0, ZROWS)])
    plsc.subcore_barrier()

    def superblock(b, carry):
        pltpu.sync_copy(src_hbm.at[wid, b], src_v)
        pltpu.sync_copy(dst_hbm.at[wid, b], dst_v)
        pltpu.sync_copy(w_hbm.at[wid, b], w_v)

        def chunk(j, c1):
            pltpu.async_copy(x_hbm.at[src_v.at[j]], rows_v, sem).wait()

            def scale(g, c2):
                wv = w_v[j, pl.ds(g * LANES, LANES)]
                for i in range(LANES):
                    e = g * LANES + i
                    wt = wv[i]
                    for k in range(D // LANES):
                        sl = pl.ds(k * LANES, LANES)
                        rows_v[e, sl] = rows_v[e, sl] * wt
                return c2

            lax.fori_loop(0, C // LANES, scale, 0)

            pltpu.sync_copy(rows_v, acc_sh.at[dst_v.at[j]], add=True)
            return c1

        lax.fori_loop(0, SB, chunk, 0)
        return carry

    lax.fori_loop(0, SBN, superblock, 0)
    plsc.subcore_barrier()

    for k in range(RPT // ZROWS):
        r0 = pl.multiple_of(sid * RPT + k * ZROWS, 8)
        pltpu.sync_copy(acc_sh.at[pl.ds(r0, ZROWS)],
                        partial_hbm.at[cid, pl.ds(r0, ZROWS)])


_spmm = pl.kernel(
    _spmm_body,
    out_type=jax.ShapeDtypeStruct((NC, NPAD, D), jnp.float32),
    mesh=plsc.VectorSubcoreMesh(core_axis_name="c", subcore_axis_name="s"),
    scratch_types=[
        pltpu.VMEM((SB, C), jnp.int32),
        pltpu.VMEM((SB, C), jnp.int32),
        pltpu.VMEM((SB, C), jnp.float32),
        pltpu.VMEM((C, D), jnp.float32),
        pltpu.VMEM((ZROWS, D), jnp.float32),
        pltpu.VMEM_SHARED((NPAD, D), jnp.float32),
        pltpu.SemaphoreType.DMA,
    ],
)

BR = 1000


def _proj_body(p_ref, w_ref, o_ref):
    s = p_ref[0] + p_ref[1]
    o_ref[...] = jnp.dot(s, w_ref[...], preferred_element_type=jnp.float32)


def _proj(partial, W):
    return pl.pallas_call(
        _proj_body,
        grid=(N // BR,),
        in_specs=[
            pl.BlockSpec((2, BR, D), lambda i: (0, i, 0)),
            pl.BlockSpec((D, D), lambda i: (0, 0)),
        ],
        out_specs=pl.BlockSpec((BR, D), lambda i: (i, 0)),
        out_shape=jax.ShapeDtypeStruct((N, D), jnp.float32),
    )(partial, W)


def kernel(input, adj_edge_index, adj_edge_weight, W):
    src = adj_edge_index[1].reshape(NW, SBN, SB, C)
    dst = adj_edge_index[0].reshape(NW, SBN, SB, C)
    wts = adj_edge_weight.reshape(NW, SBN, SB, C)
    partial = _spmm(input, src, dst, wts)
    return _proj(partial, W)


# speedup vs baseline: 9.9794x; 1.5144x over previous
import jax
import jax.numpy as jnp
from jax import lax
from jax.experimental import pallas as pl
from jax.experimental.pallas import tpu as pltpu
from jax.experimental.pallas import tpu_sc as plsc

N = 10000
E = 320000
D = 128
NC = 2
NS = 16
NW = NC * NS
LANES = 16
NPAD = 10240
RPT = NPAD // NS
C = 80
EPW = E // NW
NCHUNK = EPW // C
SB = 25
SBN = NCHUNK // SB
NPAIR = (NCHUNK - 1) // 2


def _spmm_body(x_hbm, src_hbm, dst_hbm, w_hbm, partial_hbm,
               src_v, w_v, dst_sb, rows0, rows1, acc_sh, sem_g, sem_s):
    cid = lax.axis_index("c")
    sid = lax.axis_index("s")
    wid = cid * NS + sid

    zero16 = jnp.zeros((LANES,), jnp.float32)

    def zrow(j, carry):
        for k in range(D // LANES):
            rows0[j, pl.ds(k * LANES, LANES)] = zero16
        return carry

    lax.fori_loop(0, C, zrow, 0)
    for k in range(RPT // C):
        r0 = pl.multiple_of(sid * RPT + k * C, 8)
        pltpu.sync_copy(rows0, acc_sh.at[pl.ds(r0, C)])
    plsc.subcore_barrier()

    pltpu.sync_copy(src_hbm.at[wid], src_v)
    pltpu.sync_copy(w_hbm.at[wid], w_v)

    def start_gather(t, buf):
        idx = src_v.at[pl.ds(t * C, C)]
        pltpu.async_copy(x_hbm.at[idx], buf, sem_g)

    def wait_gather(buf):
        pltpu.make_async_copy(x_hbm.at[pl.ds(0, C)], buf, sem_g).wait()

    def wait_scatter(buf):
        pltpu.make_async_copy(buf, acc_sh.at[pl.ds(0, C)], sem_s).wait()

    def scale(t, buf):
        def group(g, carry):
            wv = w_v[pl.ds(t * C + g * LANES, LANES)]
            for i in range(LANES):
                e = g * LANES + i
                wt = wv[i]
                for k in range(D // LANES):
                    sl = pl.ds(k * LANES, LANES)
                    buf[e, sl] = buf[e, sl] * wt
            return carry

        lax.fori_loop(0, C // LANES, group, 0)

    def start_scatter(t, buf):
        pltpu.async_copy(buf, acc_sh.at[dst_sb.at[t % SB]], sem_s, add=True)

    def step(t, buf_a, buf_b, first):
        wait_gather(buf_a)
        if first:
            pltpu.sync_copy(dst_hbm.at[wid, 0], dst_sb)
        else:
            wait_scatter(buf_b)

            @pl.when(t % SB == 0)
            def _():
                pltpu.sync_copy(dst_hbm.at[wid, t // SB], dst_sb)

        @pl.when(t < NCHUNK - 1)
        def _():
            start_gather(t + 1, buf_b)

        scale(t, buf_a)
        start_scatter(t, buf_a)

    start_gather(0, rows0)

    def pair(j2, carry):
        t0 = j2 * 2

        @pl.when(j2 == 0)
        def _():
            step(t0, rows0, rows1, True)

        @pl.when(j2 > 0)
        def _():
            step(t0, rows0, rows1, False)

        step(t0 + 1, rows1, rows0, False)
        return carry

    lax.fori_loop(0, NPAIR, pair, 0)
    step(NCHUNK - 1, rows0, rows1, False)
    wait_scatter(rows0)
    plsc.subcore_barrier()

    for k in range(RPT // C):
        r0 = pl.multiple_of(sid * RPT + k * C, 8)
        pltpu.sync_copy(acc_sh.at[pl.ds(r0, C)],
                        partial_hbm.at[cid, pl.ds(r0, C)])


_spmm = pl.kernel(
    _spmm_body,
    out_type=jax.ShapeDtypeStruct((NC, NPAD, D), jnp.float32),
    mesh=plsc.VectorSubcoreMesh(core_axis_name="c", subcore_axis_name="s"),
    scratch_types=[
        pltpu.VMEM((EPW,), jnp.int32),
        pltpu.VMEM((EPW,), jnp.float32),
        pltpu.VMEM((SB, C), jnp.int32),
        pltpu.VMEM((C, D), jnp.float32),
        pltpu.VMEM((C, D), jnp.float32),
        pltpu.VMEM_SHARED((NPAD, D), jnp.float32),
        pltpu.SemaphoreType.DMA,
        pltpu.SemaphoreType.DMA,
    ],
)

BR = 1000


def _proj_body(p_ref, w_ref, o_ref):
    s = p_ref[0] + p_ref[1]
    o_ref[...] = jnp.dot(s, w_ref[...], preferred_element_type=jnp.float32)


def _proj(partial, W):
    return pl.pallas_call(
        _proj_body,
        grid=(N // BR,),
        in_specs=[
            pl.BlockSpec((2, BR, D), lambda i: (0, i, 0)),
            pl.BlockSpec((D, D), lambda i: (0, 0)),
        ],
        out_specs=pl.BlockSpec((BR, D), lambda i: (i, 0)),
        out_shape=jax.ShapeDtypeStruct((N, D), jnp.float32),
    )(partial, W)


def kernel(input, adj_edge_index, adj_edge_weight, W):
    src = adj_edge_index[1].reshape(NW, EPW)
    dst = adj_edge_index[0].reshape(NW, SBN, SB, C)
    wts = adj_edge_weight.reshape(NW, EPW)
    partial = _spmm(input, src, dst, wts)
    return _proj(partial, W)
